# Initial kernel scaffold; baseline (speedup 1.0000x reference)
import functools

import jax
import jax.numpy as jnp
from jax import lax
from jax.experimental import pallas as pl
from jax.experimental.pallas import tpu as pltpu
from jax.experimental.pallas import tpu_sc as plsc

N = 10000
E = 160000
D = 256
H = 128
NTILES = 16
CHUNK = 128
PER_TILE = 10112
NCHUNKS = PER_TILE // CHUNK
EPAD = PER_TILE * NTILES
AROWS = 10240
ZROWS = AROWS // NTILES

_mesh = plsc.VectorSubcoreMesh(core_axis_name="c", subcore_axis_name="s")


@functools.partial(
    pl.kernel,
    mesh=_mesh,
    out_type=jax.ShapeDtypeStruct((2, AROWS, H), jnp.float32),
    scratch_types=[
        pltpu.VMEM_SHARED((AROWS, H), jnp.float32),
        pltpu.VMEM((NCHUNKS, CHUNK), jnp.int32),
        pltpu.VMEM((NCHUNKS, CHUNK), jnp.int32),
        pltpu.VMEM((NCHUNKS, CHUNK), jnp.int32),
        pltpu.VMEM((NCHUNKS, CHUNK), jnp.int32),
        pltpu.VMEM((CHUNK, H), jnp.float32),
        pltpu.VMEM((CHUNK, H), jnp.float32),
        pltpu.SemaphoreType.DMA,
        pltpu.SemaphoreType.DMA,
    ],
)
def _sc_message_pass(x_il, src3, dst3, a03, a13, ctab_il, out,
                     accum, src_v, dst_v, cmb_v, a1_v, rows_v, emb_v,
                     sem1, sem2):
    c = lax.axis_index("c")
    s = lax.axis_index("s")

    pltpu.sync_copy(src3.at[s], src_v)
    pltpu.sync_copy(dst3.at[s], dst_v)
    pltpu.sync_copy(a03.at[s], cmb_v)
    pltpu.sync_copy(a13.at[s], a1_v)

    def _xform(i, carry):
        for j in range(CHUNK // 16):
            sl = pl.ds(j * 16, 16)
            src_v[i, sl] = src_v[i, sl] * 2 + c
            cmb_v[i, sl] = (cmb_v[i, sl] * 3 + a1_v[i, sl]) * 2 + c
        return carry
    lax.fori_loop(0, NCHUNKS, _xform, 0)

    zero = jnp.zeros((16,), jnp.float32)
    def _zfill(i, carry):
        for j in range(H // 16):
            rows_v[i, pl.ds(j * 16, 16)] = zero
        return carry
    lax.fori_loop(0, CHUNK, _zfill, 0)
    for k in range(ZROWS // CHUNK):
        pltpu.sync_copy(rows_v, accum.at[pl.ds(s * ZROWS + k * CHUNK, CHUNK)])
    plsc.subcore_barrier()

    def _chunk(ic, carry):
        g1 = pltpu.async_copy(x_il.at[src_v.at[ic]], rows_v, sem1)
        g2 = pltpu.async_copy(ctab_il.at[cmb_v.at[ic]], emb_v, sem2)
        g1.wait()
        g2.wait()

        def _mul(i, inner):
            for j in range(H // 16):
                sl = pl.ds(j * 16, 16)
                rows_v[i, sl] = rows_v[i, sl] * emb_v[i, sl]
            return inner
        lax.fori_loop(0, CHUNK, _mul, 0)

        pltpu.sync_copy(rows_v, accum.at[dst_v.at[ic]], add=True)
        return carry
    lax.fori_loop(0, NCHUNKS, _chunk, 0)

    plsc.subcore_barrier()
    pltpu.sync_copy(accum.at[pl.ds(s * ZROWS, ZROWS)],
                    out.at[c, pl.ds(s * ZROWS, ZROWS)])


def _mlp_body(parts_ref, x_ref, cself_ref, w1_ref, b1_ref, w2_ref, b2_ref,
              out_ref):
    a = jnp.concatenate([parts_ref[0], parts_ref[1]], axis=1)
    a = a + x_ref[...] * cself_ref[...]
    h = jnp.maximum(
        jnp.dot(a, w1_ref[...], preferred_element_type=jnp.float32)
        + b1_ref[...], 0.0)
    out_ref[...] = (
        jnp.dot(h, w2_ref[...], preferred_element_type=jnp.float32)
        + b2_ref[...])


_ROWS_BLK = 1000


def kernel(x, edge_index, edge_attr, emb1, emb2, W1, b1, W2, b2):
    x_il = x.reshape(N, 2, H).reshape(2 * N, H)
    ctab = (emb1[:, None, :] + emb2[None, :, :]).reshape(-1, D)
    ctab_il = ctab.reshape(-1, 2, H).reshape(-1, H)
    cself = (emb1[4] + emb2[0]).reshape(1, D)

    pad = EPAD - E
    src3 = jnp.pad(edge_index[0], (0, pad)).reshape(NTILES, NCHUNKS, CHUNK)
    dst3 = jnp.pad(edge_index[1], (0, pad), constant_values=N).reshape(
        NTILES, NCHUNKS, CHUNK)
    a03 = jnp.pad(edge_attr[:, 0], (0, pad)).reshape(NTILES, NCHUNKS, CHUNK)
    a13 = jnp.pad(edge_attr[:, 1], (0, pad)).reshape(NTILES, NCHUNKS, CHUNK)

    parts = _sc_message_pass(x_il, src3, dst3, a03, a13, ctab_il)

    out = pl.pallas_call(
        _mlp_body,
        grid=(N // _ROWS_BLK,),
        in_specs=[
            pl.BlockSpec((2, _ROWS_BLK, H), lambda i: (0, i, 0)),
            pl.BlockSpec((_ROWS_BLK, D), lambda i: (i, 0)),
            pl.BlockSpec((1, D), lambda i: (0, 0)),
            pl.BlockSpec((D, 2 * D), lambda i: (0, 0)),
            pl.BlockSpec((1, 2 * D), lambda i: (0, 0)),
            pl.BlockSpec((2 * D, D), lambda i: (0, 0)),
            pl.BlockSpec((1, D), lambda i: (0, 0)),
        ],
        out_specs=pl.BlockSpec((_ROWS_BLK, D), lambda i: (i, 0)),
        out_shape=jax.ShapeDtypeStruct((N, D), jnp.float32),
    )(parts, x, cself, W1, b1.reshape(1, 2 * D), W2, b2.reshape(1, D))
    return out


# speedup vs baseline: 3.0219x; 3.0219x over previous
import functools

import jax
import jax.numpy as jnp
from jax import lax
from jax.experimental import pallas as pl
from jax.experimental.pallas import tpu as pltpu
from jax.experimental.pallas import tpu_sc as plsc

N = 10000
E = 160000
D = 256
H = 128
NTILES = 16
CHUNK = 128
PER_TILE = 10112
NCHUNKS = PER_TILE // CHUNK
EPAD = PER_TILE * NTILES
AROWS = 10240
ZROWS = AROWS // NTILES

_mesh = plsc.VectorSubcoreMesh(core_axis_name="c", subcore_axis_name="s")


@functools.partial(
    pl.kernel,
    mesh=_mesh,
    out_type=jax.ShapeDtypeStruct((2, AROWS, H), jnp.float32),
    scratch_types=[
        pltpu.VMEM_SHARED((AROWS, H), jnp.float32),
        pltpu.VMEM((1, CHUNK), jnp.int32),
        pltpu.VMEM((1, CHUNK), jnp.int32),
        pltpu.VMEM((1, CHUNK), jnp.int32),
        pltpu.VMEM((1, CHUNK), jnp.int32),
        pltpu.VMEM((CHUNK, H), jnp.float32),
        pltpu.VMEM((CHUNK, H), jnp.float32),
        pltpu.SemaphoreType.DMA,
        pltpu.SemaphoreType.DMA,
    ],
)
def _sc_message_pass(x_il, src3, dst3, a03, a13, ctab_il, out,
                     accum, src_v, dst_v, cmb_v, a1_v, rows_v, emb_v,
                     sem1, sem2):
    c = lax.axis_index("c")
    s = lax.axis_index("s")

    zero = jnp.zeros((16,), jnp.float32)
    def _zfill(i, carry):
        for j in range(H // 16):
            rows_v[i, pl.ds(j * 16, 16)] = zero
        return carry
    lax.fori_loop(0, CHUNK, _zfill, 0)
    for k in range(ZROWS // CHUNK):
        pltpu.sync_copy(rows_v, accum.at[pl.ds(s * ZROWS + k * CHUNK, CHUNK)])
    plsc.subcore_barrier()

    def _chunk(ic, carry):
        pltpu.sync_copy(src3.at[s, ic], src_v.at[0])
        pltpu.sync_copy(dst3.at[s, ic], dst_v.at[0])
        pltpu.sync_copy(a03.at[s, ic], cmb_v.at[0])
        pltpu.sync_copy(a13.at[s, ic], a1_v.at[0])

        for j in range(CHUNK // 16):
            sl = pl.ds(j * 16, 16)
            src_v[0, sl] = src_v[0, sl] * 2 + c
            cmb_v[0, sl] = (cmb_v[0, sl] * 3 + a1_v[0, sl]) * 2 + c

        g1 = pltpu.async_copy(x_il.at[src_v.at[0]], rows_v, sem1)
        g2 = pltpu.async_copy(ctab_il.at[cmb_v.at[0]], emb_v, sem2)
        g1.wait()
        g2.wait()

        def _mul(i, inner):
            for j in range(H // 16):
                sl = pl.ds(j * 16, 16)
                rows_v[i, sl] = rows_v[i, sl] * emb_v[i, sl]
            return inner
        lax.fori_loop(0, CHUNK, _mul, 0)

        pltpu.sync_copy(rows_v, accum.at[dst_v.at[0]], add=True)
        return carry
    lax.fori_loop(0, NCHUNKS, _chunk, 0)

    plsc.subcore_barrier()
    pltpu.sync_copy(accum.at[pl.ds(s * ZROWS, ZROWS)],
                    out.at[c, pl.ds(s * ZROWS, ZROWS)])


def _mlp_body(parts_ref, x_ref, cself_ref, w1_ref, b1_ref, w2_ref, b2_ref,
              out_ref):
    a = jnp.concatenate([parts_ref[0], parts_ref[1]], axis=1)
    a = a + x_ref[...] * cself_ref[...]
    h = jnp.maximum(
        jnp.dot(a, w1_ref[...], preferred_element_type=jnp.float32)
        + b1_ref[...], 0.0)
    out_ref[...] = (
        jnp.dot(h, w2_ref[...], preferred_element_type=jnp.float32)
        + b2_ref[...])


_ROWS_BLK = 1000


def kernel(x, edge_index, edge_attr, emb1, emb2, W1, b1, W2, b2):
    x_il = x.reshape(N, 2, H).reshape(2 * N, H)
    ctab = (emb1[:, None, :] + emb2[None, :, :]).reshape(-1, D)
    ctab_il = ctab.reshape(-1, 2, H).reshape(-1, H)
    cself = (emb1[4] + emb2[0]).reshape(1, D)

    pad = EPAD - E
    src3 = jnp.pad(edge_index[0], (0, pad)).reshape(NTILES, NCHUNKS, CHUNK)
    dst3 = jnp.pad(edge_index[1], (0, pad), constant_values=N).reshape(
        NTILES, NCHUNKS, CHUNK)
    a03 = jnp.pad(edge_attr[:, 0], (0, pad)).reshape(NTILES, NCHUNKS, CHUNK)
    a13 = jnp.pad(edge_attr[:, 1], (0, pad)).reshape(NTILES, NCHUNKS, CHUNK)

    parts = _sc_message_pass(x_il, src3, dst3, a03, a13, ctab_il)

    out = pl.pallas_call(
        _mlp_body,
        grid=(N // _ROWS_BLK,),
        in_specs=[
            pl.BlockSpec((2, _ROWS_BLK, H), lambda i: (0, i, 0)),
            pl.BlockSpec((_ROWS_BLK, D), lambda i: (i, 0)),
            pl.BlockSpec((1, D), lambda i: (0, 0)),
            pl.BlockSpec((D, 2 * D), lambda i: (0, 0)),
            pl.BlockSpec((1, 2 * D), lambda i: (0, 0)),
            pl.BlockSpec((2 * D, D), lambda i: (0, 0)),
            pl.BlockSpec((1, D), lambda i: (0, 0)),
        ],
        out_specs=pl.BlockSpec((_ROWS_BLK, D), lambda i: (i, 0)),
        out_shape=jax.ShapeDtypeStruct((N, D), jnp.float32),
    )(parts, x, cself, W1, b1.reshape(1, 2 * D), W2, b2.reshape(1, D))
    return out
